# baseline (speedup 1.0000x reference)
import functools

import jax
import jax.numpy as jnp
from jax.experimental import pallas as pl
from jax.experimental.pallas import tpu as pltpu

_BV = 2048
_NBLK_FULL = 48
_NBLK = 49
_RAG = 100000 - _NBLK_FULL * _BV
_NI = 25


def _w_copy_full(w_hbm, wbuf, wsem, j, slot):
    return pltpu.make_async_copy(
        w_hbm.at[:, pl.ds(j * _BV, _BV)], wbuf.at[slot], wsem.at[slot]
    )


def _w_copy_rag(w_hbm, rwbuf, wsem, slot):
    return pltpu.make_async_copy(
        w_hbm.at[:, pl.ds(_NBLK_FULL * _BV, _RAG)],
        rwbuf,
        wsem.at[slot],
    )


def _b_copy_full(b_hbm, bbuf, bsem, j, slot):
    return pltpu.make_async_copy(
        b_hbm.at[:, pl.ds(j * _BV, _BV)], bbuf.at[slot], bsem.at[slot]
    )


def _b_copy_rag(b_hbm, rbbuf, bsem, slot):
    return pltpu.make_async_copy(
        b_hbm.at[:, pl.ds(_NBLK_FULL * _BV, _RAG)],
        rbbuf,
        bsem.at[slot],
    )


def _start_block(w_hbm, b_hbm, wbuf, bbuf, rwbuf, rbbuf, wsem, bsem, j, slot):
    @pl.when(j < _NBLK_FULL)
    def _():
        _w_copy_full(w_hbm, wbuf, wsem, j, slot).start()
        _b_copy_full(b_hbm, bbuf, bsem, j, slot).start()

    @pl.when(j == _NBLK_FULL)
    def _():
        _w_copy_rag(w_hbm, rwbuf, wsem, slot).start()
        _b_copy_rag(b_hbm, rbbuf, bsem, slot).start()


def _proj_body(x_hbm, w_hbm, b_hbm, logits_hbm, pm_hbm, ps_hbm,
               x_v, wbuf, bbuf, rwbuf, rbbuf, robuf, obuf, m_ref, s_ref,
               wsem, bsem, osem, xsem):
    c = jax.lax.axis_index("c")
    j0 = c * _NI

    pltpu.make_async_copy(x_hbm, x_v, xsem).start()
    _start_block(w_hbm, b_hbm, wbuf, bbuf, rwbuf, rbbuf, wsem, bsem, j0, 0)
    pltpu.make_async_copy(x_hbm, x_v, xsem).wait()

    m_ref[...] = jnp.full_like(m_ref, -jnp.inf)
    s_ref[...] = jnp.zeros_like(s_ref)

    def body(i, _):
        j = j0 + i
        slot = jax.lax.rem(i, 2)

        @pl.when(j + 1 < j0 + _NI)
        def _():
            _start_block(
                w_hbm, b_hbm, wbuf, bbuf, rwbuf, rbbuf, wsem, bsem,
                j + 1, 1 - slot,
            )

        def consume(wait_w, wait_b, read_w, read_b, write_out, out_copy):
            wait_w()
            wait_b()
            logits = (
                jnp.dot(
                    x_v[...],
                    read_w(),
                    preferred_element_type=jnp.float32,
                )
                + read_b()
            )
            m_old = m_ref[:, :1]
            bm = jnp.max(logits, axis=1, keepdims=True)
            m_new = jnp.maximum(m_old, bm)
            s_new = s_ref[:, :1] * jnp.exp(m_old - m_new) + jnp.sum(
                jnp.exp(logits - m_new), axis=1, keepdims=True
            )
            m_ref[...] = jnp.broadcast_to(m_new, m_ref.shape)
            s_ref[...] = jnp.broadcast_to(s_new, s_ref.shape)

            @pl.when(i >= 2)
            def _():
                pltpu.make_async_copy(
                    obuf.at[slot],
                    logits_hbm.at[:, pl.ds((j - 2) * _BV, _BV)],
                    osem.at[slot],
                ).wait()

            write_out(logits)
            out_copy().start()

        @pl.when(j < _NBLK_FULL)
        def _():
            def wo(l):
                obuf[slot] = l

            consume(
                _w_copy_full(w_hbm, wbuf, wsem, j, slot).wait,
                _b_copy_full(b_hbm, bbuf, bsem, j, slot).wait,
                lambda: wbuf[slot],
                lambda: bbuf[slot],
                wo,
                lambda: pltpu.make_async_copy(
                    obuf.at[slot],
                    logits_hbm.at[:, pl.ds(j * _BV, _BV)],
                    osem.at[slot],
                ),
            )

        @pl.when(j == _NBLK_FULL)
        def _():
            def wo(l):
                robuf[...] = l

            consume(
                _w_copy_rag(w_hbm, rwbuf, wsem, slot).wait,
                _b_copy_rag(b_hbm, rbbuf, bsem, slot).wait,
                lambda: rwbuf[...],
                lambda: rbbuf[...],
                wo,
                lambda: pltpu.make_async_copy(
                    robuf,
                    logits_hbm.at[:, pl.ds(_NBLK_FULL * _BV, _RAG)],
                    osem.at[slot],
                ),
            )

        return 0

    jax.lax.fori_loop(0, _NI, body, 0)

    imax = jnp.minimum(_NI - 1, _NBLK - 1 - j0)

    def drain(i):
        j = j0 + i
        slot = jax.lax.rem(i, 2)

        @pl.when(j < _NBLK_FULL)
        def _():
            pltpu.make_async_copy(
                obuf.at[slot],
                logits_hbm.at[:, pl.ds(j * _BV, _BV)],
                osem.at[slot],
            ).wait()

        @pl.when(j == _NBLK_FULL)
        def _():
            pltpu.make_async_copy(
                robuf,
                logits_hbm.at[:, pl.ds(_NBLK_FULL * _BV, _RAG)],
                osem.at[slot],
            ).wait()

    drain(imax - 1)
    drain(imax)

    pltpu.sync_copy(m_ref, pm_hbm.at[c])
    pltpu.sync_copy(s_ref, ps_hbm.at[c])


def _norm_kernel(logits_ref, pm_ref, ps_ref, out_ref):
    pm = pm_ref[...]
    ps = ps_ref[...]
    m = jnp.max(pm, axis=0)[:, :1]
    s = jnp.sum(ps * jnp.exp(pm - m[None]), axis=0)[:, :1]
    out_ref[...] = jnp.exp(logits_ref[...] - m) * (1.0 / s)


def kernel(batch, W, b):
    B, S, D = batch.shape
    V = W.shape[1]
    x = batch[:, S - 1, :]
    b2 = b.reshape(1, V)

    mesh = pltpu.create_tensorcore_mesh("c", num_cores=2)

    proj = pl.kernel(
        _proj_body,
        out_type=[
            jax.ShapeDtypeStruct((B, V), jnp.float32),
            jax.ShapeDtypeStruct((2, B, 128), jnp.float32),
            jax.ShapeDtypeStruct((2, B, 128), jnp.float32),
        ],
        mesh=mesh,
        scratch_types=[
            pltpu.VMEM((B, D), jnp.float32),
            pltpu.VMEM((2, D, _BV), jnp.float32),
            pltpu.VMEM((2, 1, _BV), jnp.float32),
            pltpu.VMEM((D, _RAG), jnp.float32),
            pltpu.VMEM((1, _RAG), jnp.float32),
            pltpu.VMEM((B, _RAG), jnp.float32),
            pltpu.VMEM((2, B, _BV), jnp.float32),
            pltpu.VMEM((B, 128), jnp.float32),
            pltpu.VMEM((B, 128), jnp.float32),
            pltpu.SemaphoreType.DMA((2,)),
            pltpu.SemaphoreType.DMA((2,)),
            pltpu.SemaphoreType.DMA((2,)),
            pltpu.SemaphoreType.DMA,
        ],
        compiler_params=pltpu.CompilerParams(
            vmem_limit_bytes=60 * 1024 * 1024,
        ),
    )
    logits, pm, ps = proj(x, W, b2)

    out = pl.pallas_call(
        _norm_kernel,
        grid=(pl.cdiv(V, _BV),),
        in_specs=[
            pl.BlockSpec((B, _BV), lambda j: (0, j)),
            pl.BlockSpec((2, B, 128), lambda j: (0, 0, 0)),
            pl.BlockSpec((2, B, 128), lambda j: (0, 0, 0)),
        ],
        out_specs=pl.BlockSpec((B, _BV), lambda j: (0, j)),
        out_shape=jax.ShapeDtypeStruct((B, V), jnp.float32),
        compiler_params=pltpu.CompilerParams(
            dimension_semantics=("arbitrary",),
        ),
    )(logits, pm, ps)
    return out


# speedup vs baseline: 1.0228x; 1.0228x over previous
import functools

import jax
import jax.numpy as jnp
from jax.experimental import pallas as pl
from jax.experimental.pallas import tpu as pltpu

_KD = 32


def _proj_softmax_kernel(x_ref, w_ref, b_ref, out_ref, *, nd):
    k = pl.program_id(0)
    part = jnp.dot(x_ref[0], w_ref[...], preferred_element_type=jnp.float32)

    @pl.when(k == 0)
    def _first():
        out_ref[...] = part + b_ref[...]

    @pl.when(k != 0)
    def _acc():
        out_ref[...] = out_ref[...] + part

    @pl.when(k == nd - 1)
    def _softmax():
        m = jnp.max(out_ref[...], axis=1, keepdims=True)
        out_ref[...] = jnp.exp(out_ref[...] - m)
        s = jnp.sum(out_ref[...], axis=1, keepdims=True)
        out_ref[...] = out_ref[...] * (1.0 / s)


def kernel(batch, W, b):
    B, S, D = batch.shape
    V = W.shape[1]
    x = batch[:, S - 1, :]
    b2 = b.reshape(1, V)
    nd = D // _KD
    x3 = x.reshape(B, nd, _KD).transpose(1, 0, 2)

    out = pl.pallas_call(
        functools.partial(_proj_softmax_kernel, nd=nd),
        grid=(nd,),
        in_specs=[
            pl.BlockSpec((1, B, _KD), lambda k: (k, 0, 0)),
            pl.BlockSpec((_KD, V), lambda k: (k, 0)),
            pl.BlockSpec((1, V), lambda k: (0, 0)),
        ],
        out_specs=pl.BlockSpec((B, V), lambda k: (0, 0)),
        out_shape=jax.ShapeDtypeStruct((B, V), jnp.float32),
        compiler_params=pltpu.CompilerParams(
            dimension_semantics=("arbitrary",),
            vmem_limit_bytes=63 * 1024 * 1024,
        ),
    )(x3, W, b2)
    return out
